# Initial kernel scaffold; baseline (speedup 1.0000x reference)
import jax
import jax.numpy as jnp
from jax.experimental import pallas as pl


def kernel(e, H, r_space, e_space, action_mask, entity_emb, relation_emb, W1, b1, W2, b2):
    raise NotImplementedError("write your pallas kernel here")


# speedup vs baseline: 3.7792x; 3.7792x over previous
import functools

import jax
import jax.numpy as jnp
from jax import lax
from jax.experimental import pallas as pl
from jax.experimental.pallas import tpu as pltpu
from jax.experimental.pallas import tpu_sc as plsc

B = 4096
A = 200
A_PAD = 208
A_OUT = 256
ED = 128
HD = 256
AD = ED + HD // 2
NR_PAD = 512
HUGE = 1e9

NC, NS, L = 2, 16, 16
NW = NC * NS
BPW = B // NW

@functools.cache
def _build_gather_e():
    mesh = plsc.VectorSubcoreMesh(core_axis_name="c", subcore_axis_name="s",
                                  num_cores=NC, num_subcores=NS)

    @functools.partial(
        pl.kernel,
        out_type=jax.ShapeDtypeStruct((B, ED), jnp.float32),
        mesh=mesh,
        scratch_types=[
            pltpu.VMEM((BPW,), jnp.int32),
            pltpu.VMEM((BPW, ED), jnp.float32),
            pltpu.SemaphoreType.DMA,
        ],
        compiler_params=pltpu.CompilerParams(needs_layout_passes=False, use_tc_tiling_on_sc=False),
    )
    def _gather_e(table_hbm, idx_hbm, out_hbm, idx_v, rows_v, sem):
        wid = lax.axis_index("s") * NC + lax.axis_index("c")
        base = wid * BPW
        pltpu.sync_copy(idx_hbm.at[pl.ds(base, BPW)], idx_v)
        pltpu.async_copy(table_hbm.at[idx_v], rows_v, sem).wait()
        pltpu.sync_copy(rows_v, out_hbm.at[pl.ds(base, BPW)])

    return _gather_e


def _mlp_body(e_ref, h_ref, w1_ref, b1_ref, w2_ref, b2_ref, relT_ref,
              x2e_ref, rel_ref):
    dot = functools.partial(
        jax.lax.dot_general,
        dimension_numbers=(((1,), (0,)), ((), ())),
        preferred_element_type=jnp.float32,
        precision=jax.lax.Precision.HIGHEST,
    )
    x = dot(e_ref[...], w1_ref[:ED, :]) + dot(h_ref[...], w1_ref[ED:, :])
    x = jnp.maximum(x + b1_ref[...], 0.0)
    x2 = jnp.maximum(dot(x, w2_ref[...]) + b2_ref[...], 0.0)
    x2e_ref[...] = x2[:, ED:]
    rel_ref[...] = dot(x2[:, :ED], relT_ref[...])


def _mlp(E, H, W1, b1, W2, b2, relT):
    bs = 512
    grid = (B // bs,)
    return pl.pallas_call(
        _mlp_body,
        grid=grid,
        in_specs=[
            pl.BlockSpec((bs, ED), lambda i: (i, 0)),
            pl.BlockSpec((bs, HD), lambda i: (i, 0)),
            pl.BlockSpec((ED + HD, AD), lambda i: (0, 0)),
            pl.BlockSpec((1, AD), lambda i: (0, 0)),
            pl.BlockSpec((AD, AD), lambda i: (0, 0)),
            pl.BlockSpec((1, AD), lambda i: (0, 0)),
            pl.BlockSpec((ED, NR_PAD), lambda i: (0, 0)),
        ],
        out_specs=[
            pl.BlockSpec((bs, ED), lambda i: (i, 0)),
            pl.BlockSpec((bs, NR_PAD), lambda i: (i, 0)),
        ],
        out_shape=[
            jax.ShapeDtypeStruct((B, ED), jnp.float32),
            jax.ShapeDtypeStruct((B, NR_PAD), jnp.float32),
        ],
    )(E, H, W1, b1, W2, b2, relT)


@functools.cache
def _build_scores_sc():
    mesh = plsc.VectorSubcoreMesh(core_axis_name="c", subcore_axis_name="s",
                                  num_cores=NC, num_subcores=NS)

    @functools.partial(
        pl.kernel,
        out_type=jax.ShapeDtypeStruct((B, A_OUT), jnp.float32),
        mesh=mesh,
        scratch_types=[
            pltpu.VMEM((A_PAD,), jnp.int32),
            pltpu.VMEM((A_PAD,), jnp.int32),
            pltpu.VMEM((A_PAD,), jnp.float32),
            pltpu.VMEM((ED,), jnp.float32),
            pltpu.VMEM((NR_PAD,), jnp.float32),
            pltpu.VMEM((A_PAD, ED), jnp.float32),
            pltpu.VMEM((A_OUT,), jnp.float32),
            pltpu.SemaphoreType.DMA,
        ],
        compiler_params=pltpu.CompilerParams(needs_layout_passes=False, use_tc_tiling_on_sc=False),
    )
    def _scores_sc(esp_hbm, rsp_hbm, msk_hbm, x2e_hbm, rel_hbm, table_hbm,
                   out_hbm, eidx_v, ridx_v, msk_v, x2e_v, rel_v, rows_v, sc_v,
                   gsem):
        _scores_body(esp_hbm, rsp_hbm, msk_hbm, x2e_hbm, rel_hbm, table_hbm,
                     out_hbm, eidx_v, ridx_v, msk_v, x2e_v, rel_v, rows_v,
                     sc_v, gsem)

    return _scores_sc


def _scores_body(esp_hbm, rsp_hbm, msk_hbm, x2e_hbm, rel_hbm, table_hbm,
                 out_hbm, eidx_v, ridx_v, msk_v, x2e_v, rel_v, rows_v, sc_v,
                 gsem):
    wid = lax.axis_index("s") * NC + lax.axis_index("c")
    base = wid * BPW

    zi = jnp.zeros((L,), jnp.int32)
    zf = jnp.zeros((L,), jnp.float32)
    eidx_v[pl.ds(192, L)] = zi
    ridx_v[pl.ds(192, L)] = zi
    msk_v[pl.ds(192, L)] = zf
    for g in (13, 14, 15):
        sc_v[pl.ds(g * L, L)] = jnp.full((L,), -1e30, jnp.float32)

    def b_body(i, carry):
        b = base + i
        pltpu.sync_copy(esp_hbm.at[b], eidx_v.at[pl.ds(0, A)])
        pltpu.sync_copy(rsp_hbm.at[b], ridx_v.at[pl.ds(0, A)])
        pltpu.sync_copy(msk_hbm.at[b], msk_v.at[pl.ds(0, A)])
        pltpu.sync_copy(x2e_hbm.at[b], x2e_v)
        pltpu.sync_copy(rel_hbm.at[b], rel_v)
        c0 = pltpu.async_copy(table_hbm.at[eidx_v.at[pl.ds(0, 104)]],
                              rows_v.at[pl.ds(0, 104)], gsem)
        c1 = pltpu.async_copy(table_hbm.at[eidx_v.at[pl.ds(104, 104)]],
                              rows_v.at[pl.ds(104, 104)], gsem)
        c0.wait()
        c1.wait()

        xk = tuple(x2e_v[pl.ds(k * L, L)] for k in range(ED // L))
        lane = lax.iota(jnp.int32, L)
        perms = tuple(lane ^ sh for sh in (8, 4, 2, 1))

        def _lane_sum(v):
            for p in perms:
                v = v + v.at[p].get(mode="promise_in_bounds")
            return v

        def group_body(g, c):
            a0 = g * L
            res = jnp.zeros((L,), jnp.float32)
            for j in range(L):
                a = a0 + j
                acc = rows_v[a, pl.ds(0, L)] * xk[0]
                for k in range(1, ED // L):
                    acc = acc + rows_v[a, pl.ds(k * L, L)] * xk[k]
                res = jnp.where(lane == j, _lane_sum(acc), res)
            ri = ridx_v[pl.ds(a0, L)]
            rv = plsc.load_gather(rel_v, [ri])
            mv = msk_v[pl.ds(a0, L)]
            sc_v[pl.ds(a0, L)] = res + rv - (1.0 - mv) * HUGE
            return c

        lax.fori_loop(0, A_PAD // L, group_body, 0)
        pltpu.sync_copy(sc_v, out_hbm.at[b])
        return carry

    lax.fori_loop(0, BPW, b_body, 0)


def _soft_body(s_ref, dist_ref, ent_ref):
    s = s_ref[...]
    m = jnp.max(s, axis=1, keepdims=True)
    ex = jnp.exp(s - m)
    z = jnp.sum(ex, axis=1, keepdims=True)
    p = ex / z
    dist_ref[...] = p
    ent_ref[...] = -jnp.sum(p * jnp.log(p + 1e-20), axis=1, keepdims=True)


def _softmax(scores):
    bs = 512
    return pl.pallas_call(
        _soft_body,
        grid=(B // bs,),
        in_specs=[pl.BlockSpec((bs, A_OUT), lambda i: (i, 0))],
        out_specs=[
            pl.BlockSpec((bs, A_OUT), lambda i: (i, 0)),
            pl.BlockSpec((bs, 1), lambda i: (i, 0)),
        ],
        out_shape=[
            jax.ShapeDtypeStruct((B, A_OUT), jnp.float32),
            jax.ShapeDtypeStruct((B, 1), jnp.float32),
        ],
    )(scores)


def kernel(e, H, r_space, e_space, action_mask, entity_emb, relation_emb,
           W1, b1, W2, b2):
    e = e.astype(jnp.int32)
    r_space = r_space.astype(jnp.int32)
    e_space = e_space.astype(jnp.int32)
    nr1 = relation_emb.shape[0]
    relT = jnp.zeros((ED, NR_PAD), jnp.float32).at[:, :nr1].set(relation_emb.T)

    E = _build_gather_e()(entity_emb, e)
    x2e, rel_scores = _mlp(E, H, W1, b1.reshape(1, AD), W2, b2.reshape(1, AD),
                           relT)
    scores = _build_scores_sc()(e_space, r_space, action_mask, x2e, rel_scores,
                                entity_emb)
    dist, ent = _softmax(scores)
    return dist[:, :A], ent.reshape(B)
